# baseline (speedup 1.0000x reference)
import functools

import jax
import jax.numpy as jnp
from jax import lax
from jax.experimental import pallas as pl
from jax.experimental.pallas import tpu as pltpu
from jax.experimental.pallas import tpu_sc as plsc

N = 10000
D = 128
H = 128
K = 32
R2 = 0.15 * 0.15
NPAD = 10240
BQ = 80
BD = 400
NW = 32
CH = 80
INF = 1e30


def _proj_body(x_ref, posp_ref, w1a_ref, w1b_ref, b1_ref, g_ref, q_ref):
    q = jnp.dot(posp_ref[...], w1b_ref[...], preferred_element_type=jnp.float32)
    q_ref[...] = q
    g_ref[...] = (
        jnp.dot(x_ref[...], w1a_ref[...], preferred_element_type=jnp.float32)
        + q + b1_ref[...]
    )


def _project(x, posp, w1a, w1b, b1r):
    blk = 1000
    return pl.pallas_call(
        _proj_body,
        grid=(N // blk,),
        in_specs=[
            pl.BlockSpec((blk, D), lambda i: (i, 0)),
            pl.BlockSpec((blk, 8), lambda i: (i, 0)),
            pl.BlockSpec((D, H), lambda i: (0, 0)),
            pl.BlockSpec((8, H), lambda i: (0, 0)),
            pl.BlockSpec((1, H), lambda i: (0, 0)),
        ],
        out_specs=[
            pl.BlockSpec((blk, H), lambda i: (i, 0)),
            pl.BlockSpec((blk, H), lambda i: (i, 0)),
        ],
        out_shape=[
            jax.ShapeDtypeStruct((N, H), jnp.float32),
            jax.ShapeDtypeStruct((N, H), jnp.float32),
        ],
    )(x, posp, w1a, w1b, b1r)


_QLEV = 131071
_IMAX = 2147483647


CW = 1024
NCH = NPAD // CW


def _select_body(q_ref, p_ref, nbr_ref):
    dx = q_ref[:, 0:1] - p_ref[0:1, :]
    dy = q_ref[:, 1:2] - p_ref[1:2, :]
    dz = q_ref[:, 2:3] - p_ref[2:3, :]
    d2 = dx * dx + dy * dy + dz * dz
    valid = (d2 <= R2) & (q_ref[:, 3:4] == p_ref[3:4, :])
    colid = lax.broadcasted_iota(jnp.int32, (BQ, NPAD), 1)
    qd2 = (d2 * (65535.0 / R2)).astype(jnp.int32)
    kbits = qd2 * 16384 + colid + jnp.int32(1 << 23)
    packed = jnp.where(valid, lax.bitcast_convert_type(kbits, jnp.float32), INF)

    selfid = pl.program_id(0) * BQ + lax.broadcasted_iota(jnp.int32, (BQ, 1), 0)
    m = jnp.min(packed, axis=1, keepdims=True)
    cols = []
    for k in range(K):
        mi = lax.bitcast_convert_type(m, jnp.int32)
        cols.append(jnp.where(m < INF, jnp.bitwise_and(mi, 16383), selfid))
        if k < K - 1:
            packed = jnp.where(packed == m, INF, packed)
            m = jnp.min(packed, axis=1, keepdims=True)
    nbr_ref[...] = jnp.concatenate(cols, axis=1)


def _select(qall, prow):
    return pl.pallas_call(
        _select_body,
        grid=(N // BQ,),
        in_specs=[
            pl.BlockSpec((BQ, 8), lambda i: (i, 0)),
            pl.BlockSpec((8, NPAD), lambda i: (0, 0)),
        ],
        out_specs=pl.BlockSpec((BQ, K), lambda i: (i, 0)),
        out_shape=jax.ShapeDtypeStruct((N, K), jnp.int32),
    )(qall, prow)


_ROWS_PER_W = K * N // NW
_ITERS = _ROWS_PER_W // CH


def _gather_body(g_hbm, idx_hbm, out_hbm, idx_v, rows_v, sem):
    wid = lax.axis_index("s") * 2 + lax.axis_index("c")
    base = wid * _ROWS_PER_W

    def step(i, carry):
        off = base + i * CH
        pltpu.sync_copy(idx_hbm.at[pl.ds(off, CH)], idx_v)
        pltpu.async_copy(g_hbm.at[idx_v], rows_v, sem).wait()
        pltpu.sync_copy(rows_v, out_hbm.at[pl.ds(off, CH)])
        return carry

    lax.fori_loop(0, _ITERS, step, 0)


@functools.cache
def _make_gather():
    return pl.kernel(
        _gather_body,
        out_type=jax.ShapeDtypeStruct((K * N, H), jnp.float32),
        mesh=plsc.VectorSubcoreMesh(core_axis_name="c", subcore_axis_name="s"),
        scratch_types=[
            pltpu.VMEM((CH,), jnp.int32),
            pltpu.VMEM((CH, H), jnp.float32),
            pltpu.SemaphoreType.DMA,
        ],
    )


def _reduce_body(gg_ref, q_ref, w2_ref, b2_ref, out_ref):
    q = q_ref[...]
    acc = jnp.full((BD, H), -INF, dtype=jnp.float32)
    for k in range(K):
        p = jnp.maximum(gg_ref[k] - q, 0.0)
        h = jnp.dot(p, w2_ref[...], preferred_element_type=jnp.float32)
        acc = jnp.maximum(acc, h)
    out_ref[...] = acc + b2_ref[...]


def _reduce(gg3, q, w2, b2r):
    return pl.pallas_call(
        _reduce_body,
        grid=(N // BD,),
        in_specs=[
            pl.BlockSpec((K, BD, H), lambda i: (0, i, 0)),
            pl.BlockSpec((BD, H), lambda i: (i, 0)),
            pl.BlockSpec((H, H), lambda i: (0, 0)),
            pl.BlockSpec((1, H), lambda i: (0, 0)),
        ],
        out_specs=pl.BlockSpec((BD, H), lambda i: (i, 0)),
        out_shape=jax.ShapeDtypeStruct((N, H), jnp.float32),
    )(gg3, q, w2, b2r)


def kernel(x, pos, batch, W1, b1, W2, b2):
    w1a = W1[:D]
    w1b = jnp.zeros((8, H), jnp.float32).at[:3].set(W1[D:])
    b1r = b1.reshape(1, H)
    b2r = b2.reshape(1, H)

    batf = batch.astype(jnp.float32)
    qall = jnp.concatenate(
        [pos, batf[:, None], jnp.zeros((N, 4), jnp.float32)], axis=1)
    pad4 = jnp.broadcast_to(
        jnp.array([[1e3], [1e3], [1e3], [-1.0]], jnp.float32), (4, NPAD - N))
    prow = jnp.concatenate([
        jnp.concatenate([pos.T, batf[None, :]], axis=0),
        pad4,
    ], axis=1)
    prow = jnp.concatenate([prow, jnp.zeros((4, NPAD), jnp.float32)], axis=0)

    g, q = _project(x, qall, w1a, w1b, b1r)
    nbr = _select(qall, prow)

    idx_flat = jnp.transpose(nbr).reshape(-1)
    gg = _make_gather()(g, idx_flat)
    gg3 = gg.reshape(K, N, H)

    out_x = _reduce(gg3, q, W2, b2r)
    return out_x, pos, batch


# speedup vs baseline: 1.0597x; 1.0597x over previous
import functools

import jax
import jax.numpy as jnp
from jax import lax
from jax.experimental import pallas as pl
from jax.experimental.pallas import tpu as pltpu
from jax.experimental.pallas import tpu_sc as plsc

N = 10000
D = 128
H = 128
K = 32
R2 = 0.15 * 0.15
NPAD = 10240
BQ = 80
BD = 400
NW = 32
CH = 80
INF = 1e30


def _proj_body(x_ref, posp_ref, w1a_ref, w1b_ref, b1_ref, g_ref, q_ref):
    q = jnp.dot(posp_ref[...], w1b_ref[...], preferred_element_type=jnp.float32)
    q_ref[...] = q
    g_ref[...] = (
        jnp.dot(x_ref[...], w1a_ref[...], preferred_element_type=jnp.float32)
        + q + b1_ref[...]
    )


def _project(x, posp, w1a, w1b, b1r):
    blk = 1000
    return pl.pallas_call(
        _proj_body,
        grid=(N // blk,),
        in_specs=[
            pl.BlockSpec((blk, D), lambda i: (i, 0)),
            pl.BlockSpec((blk, 8), lambda i: (i, 0)),
            pl.BlockSpec((D, H), lambda i: (0, 0)),
            pl.BlockSpec((8, H), lambda i: (0, 0)),
            pl.BlockSpec((1, H), lambda i: (0, 0)),
        ],
        out_specs=[
            pl.BlockSpec((blk, H), lambda i: (i, 0)),
            pl.BlockSpec((blk, H), lambda i: (i, 0)),
        ],
        out_shape=[
            jax.ShapeDtypeStruct((N, H), jnp.float32),
            jax.ShapeDtypeStruct((N, H), jnp.float32),
        ],
    )(x, posp, w1a, w1b, b1r)


_QLEV = 131071
_IMAX = 2147483647


CW = 1024
NCH = NPAD // CW


def _select_body(row0, q_ref, p_ref, nbr_ref):
    dx = q_ref[:, 0:1] - p_ref[0:1, :]
    dy = q_ref[:, 1:2] - p_ref[1:2, :]
    dz = q_ref[:, 2:3] - p_ref[2:3, :]
    d2 = dx * dx + dy * dy + dz * dz
    valid = (d2 <= R2) & (q_ref[:, 3:4] == p_ref[3:4, :])
    colid = lax.broadcasted_iota(jnp.int32, (BQ, NPAD), 1)
    qd2 = (d2 * (65535.0 / R2)).astype(jnp.int32)
    kbits = qd2 * 16384 + colid + jnp.int32(1 << 23)
    packed = jnp.where(valid, lax.bitcast_convert_type(kbits, jnp.float32), INF)

    selfid = (row0 + pl.program_id(0) * BQ
              + lax.broadcasted_iota(jnp.int32, (BQ, 1), 0))
    m = jnp.min(packed, axis=1, keepdims=True)
    cols = []
    for k in range(K):
        mi = lax.bitcast_convert_type(m, jnp.int32)
        cols.append(jnp.where(m < INF, jnp.bitwise_and(mi, 16383), selfid))
        if k < K - 1:
            packed = jnp.where(packed == m, INF, packed)
            m = jnp.min(packed, axis=1, keepdims=True)
    nbr_ref[...] = jnp.concatenate(cols, axis=1)


def _select(qall_h, prow, row0, nh):
    return pl.pallas_call(
        functools.partial(_select_body, row0),
        grid=(nh // BQ,),
        in_specs=[
            pl.BlockSpec((BQ, 8), lambda i: (i, 0)),
            pl.BlockSpec((8, NPAD), lambda i: (0, 0)),
        ],
        out_specs=pl.BlockSpec((BQ, K), lambda i: (i, 0)),
        out_shape=jax.ShapeDtypeStruct((nh, K), jnp.int32),
    )(qall_h, prow)


def _gather_body(rows_per_w, g_hbm, idx_hbm, out_hbm, idx_v, rows_v, sem):
    wid = lax.axis_index("s") * 2 + lax.axis_index("c")
    base = wid * rows_per_w

    def step(i, carry):
        off = base + i * CH
        pltpu.sync_copy(idx_hbm.at[pl.ds(off, CH)], idx_v)
        pltpu.async_copy(g_hbm.at[idx_v], rows_v, sem).wait()
        pltpu.sync_copy(rows_v, out_hbm.at[pl.ds(off, CH)])
        return carry

    lax.fori_loop(0, rows_per_w // CH, step, 0)


@functools.cache
def _make_gather(nh):
    return pl.kernel(
        functools.partial(_gather_body, K * nh // NW),
        out_type=jax.ShapeDtypeStruct((K * nh, H), jnp.float32),
        mesh=plsc.VectorSubcoreMesh(core_axis_name="c", subcore_axis_name="s"),
        scratch_types=[
            pltpu.VMEM((CH,), jnp.int32),
            pltpu.VMEM((CH, H), jnp.float32),
            pltpu.SemaphoreType.DMA,
        ],
    )


def _reduce_body(gg_ref, q_ref, w2_ref, b2_ref, out_ref):
    q = q_ref[...]
    acc = jnp.full((BD, H), -INF, dtype=jnp.float32)
    for k in range(K):
        p = jnp.maximum(gg_ref[k] - q, 0.0)
        h = jnp.dot(p, w2_ref[...], preferred_element_type=jnp.float32)
        acc = jnp.maximum(acc, h)
    out_ref[...] = acc + b2_ref[...]


def _reduce(gg3, q, w2, b2r):
    nh = q.shape[0]
    return pl.pallas_call(
        _reduce_body,
        grid=(nh // BD,),
        in_specs=[
            pl.BlockSpec((K, BD, H), lambda i: (0, i, 0)),
            pl.BlockSpec((BD, H), lambda i: (i, 0)),
            pl.BlockSpec((H, H), lambda i: (0, 0)),
            pl.BlockSpec((1, H), lambda i: (0, 0)),
        ],
        out_specs=pl.BlockSpec((BD, H), lambda i: (i, 0)),
        out_shape=jax.ShapeDtypeStruct((nh, H), jnp.float32),
    )(gg3, q, w2, b2r)


def kernel(x, pos, batch, W1, b1, W2, b2):
    w1a = W1[:D]
    w1b = jnp.zeros((8, H), jnp.float32).at[:3].set(W1[D:])
    b1r = b1.reshape(1, H)
    b2r = b2.reshape(1, H)

    batf = batch.astype(jnp.float32)
    qall = jnp.concatenate(
        [pos, batf[:, None], jnp.zeros((N, 4), jnp.float32)], axis=1)
    pad4 = jnp.broadcast_to(
        jnp.array([[1e3], [1e3], [1e3], [-1.0]], jnp.float32), (4, NPAD - N))
    prow = jnp.concatenate([
        jnp.concatenate([pos.T, batf[None, :]], axis=0),
        pad4,
    ], axis=1)
    prow = jnp.concatenate([prow, jnp.zeros((4, NPAD), jnp.float32)], axis=0)

    g, q = _project(x, qall, w1a, w1b, b1r)

    outs = []
    for row0, nh in ((0, 4800), (4800, 5200)):
        nbr = _select(qall[row0:row0 + nh], prow, row0, nh)
        idx_flat = jnp.transpose(nbr).reshape(-1)
        gg = _make_gather(nh)(g, idx_flat)
        outs.append(_reduce(gg.reshape(K, nh, H), q[row0:row0 + nh], W2, b2r))
    out_x = jnp.concatenate(outs, axis=0)
    return out_x, pos, batch


# speedup vs baseline: 1.0599x; 1.0002x over previous
import functools

import jax
import jax.numpy as jnp
from jax import lax
from jax.experimental import pallas as pl
from jax.experimental.pallas import tpu as pltpu
from jax.experimental.pallas import tpu_sc as plsc

N = 10000
D = 128
H = 128
K = 32
R2 = 0.15 * 0.15
NPAD = 10240
BQ = 80
BD = 400
NW = 32
CH = 80
INF = 1e30


def _proj_body(x_ref, posp_ref, w1a_ref, w1b_ref, b1_ref, g_ref, q_ref):
    q = jnp.dot(posp_ref[...], w1b_ref[...], preferred_element_type=jnp.float32)
    q_ref[...] = q
    g_ref[...] = (
        jnp.dot(x_ref[...], w1a_ref[...], preferred_element_type=jnp.float32)
        + q + b1_ref[...]
    )


def _project(x, posp, w1a, w1b, b1r):
    blk = 1000
    return pl.pallas_call(
        _proj_body,
        grid=(N // blk,),
        in_specs=[
            pl.BlockSpec((blk, D), lambda i: (i, 0)),
            pl.BlockSpec((blk, 8), lambda i: (i, 0)),
            pl.BlockSpec((D, H), lambda i: (0, 0)),
            pl.BlockSpec((8, H), lambda i: (0, 0)),
            pl.BlockSpec((1, H), lambda i: (0, 0)),
        ],
        out_specs=[
            pl.BlockSpec((blk, H), lambda i: (i, 0)),
            pl.BlockSpec((blk, H), lambda i: (i, 0)),
        ],
        out_shape=[
            jax.ShapeDtypeStruct((N, H), jnp.float32),
            jax.ShapeDtypeStruct((N, H), jnp.float32),
        ],
    )(x, posp, w1a, w1b, b1r)


def _select_body(row0, q_ref, p_ref, nbr_ref):
    dx = q_ref[:, 0:1] - p_ref[0:1, :]
    dy = q_ref[:, 1:2] - p_ref[1:2, :]
    dz = q_ref[:, 2:3] - p_ref[2:3, :]
    d2 = dx * dx + dy * dy + dz * dz
    valid = (d2 <= R2) & (q_ref[:, 3:4] == p_ref[3:4, :])
    colid = lax.broadcasted_iota(jnp.int32, (BQ, NPAD), 1)
    qd2 = (d2 * (65535.0 / R2)).astype(jnp.int32)
    kbits = qd2 * 16384 + colid + jnp.int32(1 << 23)
    packed = jnp.where(valid, lax.bitcast_convert_type(kbits, jnp.float32), INF)

    selfid = (row0 + pl.program_id(0) * BQ
              + lax.broadcasted_iota(jnp.int32, (BQ, 1), 0))
    m = jnp.min(packed, axis=1, keepdims=True)
    cols = []
    for k in range(K):
        mi = lax.bitcast_convert_type(m, jnp.int32)
        cols.append(jnp.where(m < INF, jnp.bitwise_and(mi, 16383), selfid))
        if k < K - 1:
            packed = jnp.where(packed == m, INF, packed)
            m = jnp.min(packed, axis=1, keepdims=True)
    nbr_ref[...] = jnp.concatenate(cols, axis=1)


def _select(qall_h, prow, row0, nh):
    return pl.pallas_call(
        functools.partial(_select_body, row0),
        grid=(nh // BQ,),
        in_specs=[
            pl.BlockSpec((BQ, 8), lambda i: (i, 0)),
            pl.BlockSpec((8, NPAD), lambda i: (0, 0)),
        ],
        out_specs=pl.BlockSpec((BQ, K), lambda i: (i, 0)),
        out_shape=jax.ShapeDtypeStruct((nh, K), jnp.int32),
    )(qall_h, prow)


def _gather_body(rows_per_w, g_hbm, idx_hbm, out_hbm, idx_v, rows_v, sem):
    wid = lax.axis_index("s") * 2 + lax.axis_index("c")
    base = wid * rows_per_w

    def step(i, carry):
        off = base + i * CH
        pltpu.sync_copy(idx_hbm.at[pl.ds(off, CH)], idx_v)
        pltpu.async_copy(g_hbm.at[idx_v], rows_v, sem).wait()
        pltpu.sync_copy(rows_v, out_hbm.at[pl.ds(off, CH)])
        return carry

    lax.fori_loop(0, rows_per_w // CH, step, 0)


@functools.cache
def _make_gather(nh):
    return pl.kernel(
        functools.partial(_gather_body, K * nh // NW),
        out_type=jax.ShapeDtypeStruct((K * nh, H), jnp.float32),
        mesh=plsc.VectorSubcoreMesh(core_axis_name="c", subcore_axis_name="s"),
        scratch_types=[
            pltpu.VMEM((CH,), jnp.int32),
            pltpu.VMEM((CH, H), jnp.float32),
            pltpu.SemaphoreType.DMA,
        ],
    )


def _reduce_body(gg_ref, q_ref, w2_ref, b2_ref, out_ref):
    q = q_ref[...]
    acc = jnp.full((BD, H), -INF, dtype=jnp.float32)
    for k in range(K):
        p = jnp.maximum(gg_ref[k] - q, 0.0)
        h = jnp.dot(p, w2_ref[...], preferred_element_type=jnp.float32)
        acc = jnp.maximum(acc, h)
    out_ref[...] = acc + b2_ref[...]


def _reduce(gg3, q, w2, b2r):
    nh = q.shape[0]
    return pl.pallas_call(
        _reduce_body,
        grid=(nh // BD,),
        in_specs=[
            pl.BlockSpec((K, BD, H), lambda i: (0, i, 0)),
            pl.BlockSpec((BD, H), lambda i: (i, 0)),
            pl.BlockSpec((H, H), lambda i: (0, 0)),
            pl.BlockSpec((1, H), lambda i: (0, 0)),
        ],
        out_specs=pl.BlockSpec((BD, H), lambda i: (i, 0)),
        out_shape=jax.ShapeDtypeStruct((nh, H), jnp.float32),
    )(gg3, q, w2, b2r)


def kernel(x, pos, batch, W1, b1, W2, b2):
    w1a = W1[:D]
    w1b = jnp.zeros((8, H), jnp.float32).at[:3].set(W1[D:])
    b1r = b1.reshape(1, H)
    b2r = b2.reshape(1, H)

    batf = batch.astype(jnp.float32)
    qall = jnp.concatenate(
        [pos, batf[:, None], jnp.zeros((N, 4), jnp.float32)], axis=1)
    pad4 = jnp.broadcast_to(
        jnp.array([[1e3], [1e3], [1e3], [-1.0]], jnp.float32), (4, NPAD - N))
    prow = jnp.concatenate([
        jnp.concatenate([pos.T, batf[None, :]], axis=0),
        pad4,
    ], axis=1)
    prow = jnp.concatenate([prow, jnp.zeros((4, NPAD), jnp.float32)], axis=0)

    g, q = _project(x, qall, w1a, w1b, b1r)

    outs = []
    for row0, nh in ((0, 4800), (4800, 5200)):
        nbr = _select(qall[row0:row0 + nh], prow, row0, nh)
        idx_flat = jnp.transpose(nbr).reshape(-1)
        gg = _make_gather(nh)(g, idx_flat)
        outs.append(_reduce(gg.reshape(K, nh, H), q[row0:row0 + nh], W2, b2r))
    out_x = jnp.concatenate(outs, axis=0)
    return out_x, pos, batch
